# baseline (speedup 1.0000x reference)
import functools

import jax
import jax.numpy as jnp
from jax import lax
from jax.experimental import pallas as pl
from jax.experimental.pallas import tpu as pltpu
from jax.experimental.pallas import tpu_sc as plsc

_VOCAB = 1000000
_EMBED_DIM = 32
_BATCH = 16384
_HIST = 50

_info = plsc.get_sparse_core_info()
_NC, _NS_SUB = _info.num_cores, _info.num_subcores
_NW = _NC * _NS_SUB
_TB = 128
_NT = _BATCH // _TB
_TPW = _NT // _NW


def _make_kernel():
    mesh = plsc.VectorSubcoreMesh(core_axis_name="c", subcore_axis_name="s")

    @functools.partial(
        pl.kernel,
        out_type=jax.ShapeDtypeStruct(
            (_HIST, _EMBED_DIM // 8, _NT, 8, 128), jnp.float32
        ),
        mesh=mesh,
        scratch_types=(
            [pltpu.VMEM((_TB, _HIST), jnp.int32)]
            + [pltpu.VMEM((_TB,), jnp.int32) for _ in range(2)]
            + [pltpu.VMEM((_TB, _EMBED_DIM), jnp.float32) for _ in range(2)]
            + [pltpu.VMEM((_EMBED_DIM // 8, 8, 128), jnp.float32) for _ in range(2)]
            + [pltpu.SemaphoreType.DMA for _ in range(4)]
        ),
        compiler_params=pltpu.CompilerParams(
            use_tc_tiling_on_sc=False, needs_layout_passes=False
        ),
    )
    def gather_kernel(table_hbm, idx_hbm, out_hbm, *scratch):
        idx_tile = scratch[0]
        il = scratch[1:3]
        rows_in = scratch[3:5]
        rows_t = scratch[5:7]
        gsems = scratch[7:9]
        osems = scratch[9:11]
        wid = lax.axis_index("s") * _NC + lax.axis_index("c")
        iota16 = lax.iota(jnp.int32, 16)

        def build_il(s, h):
            hvec = jnp.full((16,), h, jnp.int32)
            for j in range(8):
                v = plsc.load_gather(idx_tile, [j * 16 + iota16, hvec])
                il[s][pl.ds(j * 16, 16)] = v

        def gather(s):
            return pltpu.make_async_copy(table_hbm.at[il[s]], rows_in[s], gsems[s])

        def transpose(s):
            def tr(r, carry):
                a = r // 8
                r8 = r % 8
                rvec = jnp.full((16,), r, jnp.int32)
                for j in range(8):
                    v = plsc.load_gather(rows_in[s], [j * 16 + iota16, rvec])
                    rows_t[s][a, r8, pl.ds(j * 16, 16)] = v
                return carry

            lax.fori_loop(0, _EMBED_DIM, tr, 0)

        def out_copies(s, h, t):
            return [
                pltpu.make_async_copy(
                    rows_t[s].at[a], out_hbm.at[h, a, t], osems[s]
                )
                for a in range(_EMBED_DIM // 8)
            ]

        def start4(s, h, t):
            for c in out_copies(s, h, t):
                c.start()

        def wait4(s, h, t):
            for c in out_copies(s, h, t):
                c.wait()

        for ti in range(_TPW):
            t = wid * _TPW + ti
            pltpu.sync_copy(idx_hbm.at[pl.ds(t * _TB, _TB)], idx_tile)
            build_il(0, 0)
            gather(0).start()
            build_il(1, 1)
            gather(1).start()
            gather(0).wait()
            transpose(0)
            start4(0, 0, t)
            gather(1).wait()
            transpose(1)
            start4(1, 1, t)

            def pair_body(i, carry):
                h0 = 2 * i + 2
                h1 = 2 * i + 3
                wait4(0, h0 - 2, t)
                build_il(0, h0)
                gather(0).start()
                wait4(1, h1 - 2, t)
                build_il(1, h1)
                gather(1).start()
                gather(0).wait()
                transpose(0)
                start4(0, h0, t)
                gather(1).wait()
                transpose(1)
                start4(1, h1, t)
                return carry

            lax.fori_loop(0, (_HIST - 2) // 2, pair_body, 0)
            wait4(0, _HIST - 2, t)
            wait4(1, _HIST - 1, t)

    return gather_kernel


_gather = _make_kernel()


def kernel(indices, weight):
    out5 = _gather(weight, indices.astype(jnp.int32))
    return out5.transpose(2, 4, 0, 1, 3).reshape(_BATCH, _HIST, _EMBED_DIM)
